# Initial kernel scaffold; baseline (speedup 1.0000x reference)
import functools
import numpy as np
import jax
import jax.numpy as jnp
from jax import lax
from jax.experimental import pallas as pl
from jax.experimental.pallas import tpu as pltpu
from jax.experimental.pallas import tpu_sc as plsc

B = 4096
DENSE_IN = 13
F = 26
VOCAB = 100000
D = 32
N1 = F + 1
NPAD = 32
BF = B * F

NW = 32
ROWS_PER_W = BF // NW
CHUNK = 128
NCH = ROWS_PER_W // CHUNK

BBLK = 256
NBLK = B // BBLK

_PI, _PJ = np.triu_indices(N1, k=1)


def _sc_gather(table_flat, idx2d):
    mesh = plsc.VectorSubcoreMesh(core_axis_name="c", subcore_axis_name="s")

    @functools.partial(
        pl.kernel,
        mesh=mesh,
        out_type=jax.ShapeDtypeStruct((BF, D), jnp.float32),
        scratch_types=[
            pltpu.VMEM((NCH, CHUNK), jnp.int32),
            pltpu.VMEM((ROWS_PER_W, D), jnp.float32),
            pltpu.SemaphoreType.DMA,
        ],
    )
    def gather_kernel(table_hbm, idx_hbm, out_hbm, idx_v, rows_v, sem):
        wid = lax.axis_index("s") * 2 + lax.axis_index("c")
        pltpu.sync_copy(idx_hbm.at[pl.ds(wid * NCH, NCH)], idx_v)
        copies = []
        for j in range(NCH):
            copies.append(
                pltpu.async_copy(
                    table_hbm.at[idx_v.at[j]],
                    rows_v.at[pl.ds(j * CHUNK, CHUNK)],
                    sem,
                )
            )
        for c in copies:
            c.wait()
        pltpu.sync_copy(rows_v, out_hbm.at[pl.ds(wid * ROWS_PER_W, ROWS_PER_W)])

    return gather_kernel(table_flat, idx2d)


def _tc_body(dense_ref, pooled_ref, lab_ref,
             dW0_ref, dB0_ref, dW1_ref, dB1_ref, dW2_ref, dB2_ref,
             oW0a_ref, Wp_ref, oB0_ref, oW1_ref, oB1_ref, oW2_ref, oB2_ref,
             probs_ref, loss_ref):
    x = dense_ref[...]
    x = jnp.maximum(x @ dW0_ref[...] + dB0_ref[...], 0.0)
    x = jnp.maximum(x @ dW1_ref[...] + dB1_ref[...], 0.0)
    e = jnp.maximum(x @ dW2_ref[...] + dB2_ref[...], 0.0)

    comb = jnp.concatenate([e, pooled_ref[...]], axis=1)
    P = comb.reshape(BBLK, N1, D)
    Ppad = jnp.concatenate(
        [P, jnp.zeros((BBLK, NPAD - N1, D), jnp.float32)], axis=1)

    cols = []
    for n in range(N1):
        an = P[:, n:n + 1, :]
        cols.append(jnp.sum(Ppad * an, axis=2))
    flat = jnp.concatenate(cols, axis=1)

    t = e @ oW0a_ref[...] + flat @ Wp_ref[...] + oB0_ref[...]
    h = jnp.maximum(t, 0.0)
    h = jnp.maximum(h @ oW1_ref[...] + oB1_ref[...], 0.0)
    z = h @ oW2_ref[...] + oB2_ref[...]

    pr = 1.0 / (1.0 + jnp.exp(-z))
    probs_ref[...] = pr

    p = jnp.clip(pr, 1e-7, 1.0 - 1e-7)
    lab = lab_ref[...]
    s = jnp.sum(lab * jnp.log(p) + (1.0 - lab) * jnp.log1p(-p))
    contrib = -s * (1.0 / B)

    i = pl.program_id(0)

    @pl.when(i == 0)
    def _():
        loss_ref[0, 0] = contrib

    @pl.when(i > 0)
    def _():
        loss_ref[0, 0] += contrib


def _tc_forward(dense, pooled2d, labf, dW0, dB0, dW1, dB1, dW2, dB2,
                oW0a, Wp, oB0, oW1, oB1, oW2, oB2, interpret=False):
    full = lambda a: pl.BlockSpec(a.shape, lambda i: (0, 0))
    grid = (NBLK,)
    probs, loss = pl.pallas_call(
        _tc_body,
        grid=grid,
        in_specs=[
            pl.BlockSpec((BBLK, DENSE_IN), lambda i: (i, 0)),
            pl.BlockSpec((BBLK, F * D), lambda i: (i, 0)),
            pl.BlockSpec((BBLK, 1), lambda i: (i, 0)),
            full(dW0), full(dB0), full(dW1), full(dB1), full(dW2), full(dB2),
            full(oW0a), full(Wp), full(oB0), full(oW1), full(oB1),
            full(oW2), full(oB2),
        ],
        out_specs=[
            pl.BlockSpec((BBLK, 1), lambda i: (i, 0)),
            pl.BlockSpec((1, 1), lambda i: (0, 0)),
        ],
        out_shape=[
            jax.ShapeDtypeStruct((B, 1), jnp.float32),
            jax.ShapeDtypeStruct((1, 1), jnp.float32),
        ],
        interpret=interpret,
    )(dense, pooled2d, labf, dW0, dB0, dW1, dB1, dW2, dB2,
      oW0a, Wp, oB0, oW1, oB1, oW2, oB2)
    return probs, loss


def _fold_pair_weights(oW0):
    H = oW0.shape[1]
    oW0p = oW0[D:]
    W2 = jnp.zeros((N1, N1, H), jnp.float32)
    W2 = W2.at[_PI, _PJ].set(0.5 * oW0p).at[_PJ, _PI].set(0.5 * oW0p)
    W2 = jnp.pad(W2, ((0, 0), (0, NPAD - N1), (0, 0)))
    return W2.reshape(N1 * NPAD, H)


def kernel(dense_features, sparse_indices, labels, tables,
           dW0, dB0, dW1, dB1, dW2, dB2,
           oW0, oB0, oW1, oB1, oW2, oB2):
    table_flat = tables.reshape(F * VOCAB, D)
    flat_idx = (sparse_indices
                + (jnp.arange(F, dtype=jnp.int32) * VOCAB)[None, :])
    idx2d = flat_idx.reshape(BF // CHUNK, CHUNK)

    pooled = _sc_gather(table_flat, idx2d)
    pooled2d = pooled.reshape(B, F * D)

    Wp = _fold_pair_weights(oW0)
    probs, loss = _tc_forward(
        dense_features, pooled2d,
        labels.astype(jnp.float32).reshape(B, 1),
        dW0, dB0.reshape(1, -1), dW1, dB1.reshape(1, -1),
        dW2, dB2.reshape(1, -1),
        oW0[:D], Wp, oB0.reshape(1, -1), oW1, oB1.reshape(1, -1),
        oW2, oB2.reshape(1, 1))
    return (loss[0, 0], probs.reshape(B), labels)


# speedup vs baseline: 1.6174x; 1.6174x over previous
import functools
import numpy as np
import jax
import jax.numpy as jnp
from jax import lax
from jax.experimental import pallas as pl
from jax.experimental.pallas import tpu as pltpu
from jax.experimental.pallas import tpu_sc as plsc

B = 4096
DENSE_IN = 13
F = 26
VOCAB = 100000
D = 32
N1 = F + 1
NPAD = 32
BF = B * F

NW = 32
ROWS_PER_W = BF // NW
CHUNK = 128
NCH = ROWS_PER_W // CHUNK

BBLK = 256
NBLK = B // BBLK

_PI, _PJ = np.triu_indices(N1, k=1)


def _sc_gather(table_flat, idx2d):
    mesh = plsc.VectorSubcoreMesh(core_axis_name="c", subcore_axis_name="s")

    @functools.partial(
        pl.kernel,
        mesh=mesh,
        out_type=jax.ShapeDtypeStruct((BF, D), jnp.float32),
        scratch_types=[
            pltpu.VMEM((NCH, CHUNK), jnp.int32),
            pltpu.VMEM((ROWS_PER_W, D), jnp.float32),
            pltpu.SemaphoreType.DMA,
        ],
        compiler_params=pltpu.CompilerParams(use_tc_tiling_on_sc=False),
    )
    def gather_kernel(table_hbm, idx_hbm, out_hbm, idx_v, rows_v, sem):
        wid = lax.axis_index("s") * 2 + lax.axis_index("c")
        pltpu.sync_copy(idx_hbm.at[wid], idx_v)
        copies = []
        for j in range(NCH):
            copies.append(
                pltpu.async_copy(
                    table_hbm.at[idx_v.at[j]],
                    rows_v.at[pl.ds(j * CHUNK, CHUNK)],
                    sem,
                )
            )
        for c in copies:
            c.wait()
        pltpu.sync_copy(rows_v, out_hbm.at[pl.ds(wid * ROWS_PER_W, ROWS_PER_W)])

    return gather_kernel(table_flat, idx2d)


def _tc_body(dense_ref, pooled_ref, lab_ref,
             dW0_ref, dB0_ref, dW1_ref, dB1_ref, dW2_ref, dB2_ref,
             oW0a_ref, Wp_ref, oB0_ref, oW1_ref, oB1_ref, oW2_ref, oB2_ref,
             probs_ref, loss_ref):
    x = dense_ref[...]
    x = jnp.maximum(x @ dW0_ref[...] + dB0_ref[...], 0.0)
    x = jnp.maximum(x @ dW1_ref[...] + dB1_ref[...], 0.0)
    e = jnp.maximum(x @ dW2_ref[...] + dB2_ref[...], 0.0)

    comb = jnp.concatenate([e, pooled_ref[...]], axis=1)
    P = comb.reshape(BBLK, N1, D)
    Ppad = jnp.concatenate(
        [P, jnp.zeros((BBLK, NPAD - N1, D), jnp.float32)], axis=1)

    cols = []
    for n in range(N1):
        an = P[:, n:n + 1, :]
        cols.append(jnp.sum(Ppad * an, axis=2))
    flat = jnp.concatenate(cols, axis=1)

    t = e @ oW0a_ref[...] + flat @ Wp_ref[...] + oB0_ref[...]
    h = jnp.maximum(t, 0.0)
    h = jnp.maximum(h @ oW1_ref[...] + oB1_ref[...], 0.0)
    z = h @ oW2_ref[...] + oB2_ref[...]

    pr = 1.0 / (1.0 + jnp.exp(-z))
    probs_ref[...] = pr

    p = jnp.clip(pr, 1e-7, 1.0 - 1e-7)
    lab = lab_ref[...]
    s = jnp.sum(lab * jnp.log(p) + (1.0 - lab) * jnp.log1p(-p),
                keepdims=True)
    contrib = -s * (1.0 / B)

    i = pl.program_id(0)

    @pl.when(i == 0)
    def _():
        loss_ref[...] = contrib

    @pl.when(i > 0)
    def _():
        loss_ref[...] += contrib


def _tc_forward(dense, pooled2d, labf, dW0, dB0, dW1, dB1, dW2, dB2,
                oW0a, Wp, oB0, oW1, oB1, oW2, oB2, interpret=False):
    full = lambda a: pl.BlockSpec(a.shape, lambda i: (0, 0))
    grid = (NBLK,)
    probs, loss = pl.pallas_call(
        _tc_body,
        grid=grid,
        in_specs=[
            pl.BlockSpec((BBLK, DENSE_IN), lambda i: (i, 0)),
            pl.BlockSpec((BBLK, F * D), lambda i: (i, 0)),
            pl.BlockSpec((BBLK, 1), lambda i: (i, 0)),
            full(dW0), full(dB0), full(dW1), full(dB1), full(dW2), full(dB2),
            full(oW0a), full(Wp), full(oB0), full(oW1), full(oB1),
            full(oW2), full(oB2),
        ],
        out_specs=[
            pl.BlockSpec((BBLK, 1), lambda i: (i, 0)),
            pl.BlockSpec((1, 1), lambda i: (0, 0)),
        ],
        out_shape=[
            jax.ShapeDtypeStruct((B, 1), jnp.float32),
            jax.ShapeDtypeStruct((1, 1), jnp.float32),
        ],
        interpret=interpret,
    )(dense, pooled2d, labf, dW0, dB0, dW1, dB1, dW2, dB2,
      oW0a, Wp, oB0, oW1, oB1, oW2, oB2)
    return probs, loss


def _fold_pair_weights(oW0):
    H = oW0.shape[1]
    oW0p = oW0[D:]
    W2 = jnp.zeros((N1, N1, H), jnp.float32)
    W2 = W2.at[_PI, _PJ].set(0.5 * oW0p).at[_PJ, _PI].set(0.5 * oW0p)
    W2 = jnp.pad(W2, ((0, 0), (0, NPAD - N1), (0, 0)))
    return W2.reshape(N1 * NPAD, H)


def kernel(dense_features, sparse_indices, labels, tables,
           dW0, dB0, dW1, dB1, dW2, dB2,
           oW0, oB0, oW1, oB1, oW2, oB2):
    table_flat = tables.reshape(F * VOCAB, D)
    flat_idx = (sparse_indices
                + (jnp.arange(F, dtype=jnp.int32) * VOCAB)[None, :])
    idx2d = flat_idx.reshape(NW, NCH, CHUNK)

    pooled = _sc_gather(table_flat, idx2d)
    pooled2d = pooled.reshape(B, F * D)

    Wp = _fold_pair_weights(oW0)
    probs, loss = _tc_forward(
        dense_features, pooled2d,
        labels.astype(jnp.float32).reshape(B, 1),
        dW0, dB0.reshape(1, -1), dW1, dB1.reshape(1, -1),
        dW2, dB2.reshape(1, -1),
        oW0[:D], Wp, oB0.reshape(1, -1), oW1, oB1.reshape(1, -1),
        oW2, oB2.reshape(1, 1))
    return (loss[0, 0], probs.reshape(B), labels)


# speedup vs baseline: 2.1855x; 1.3513x over previous
import functools
import numpy as np
import jax
import jax.numpy as jnp
from jax import lax
from jax.experimental import pallas as pl
from jax.experimental.pallas import tpu as pltpu
from jax.experimental.pallas import tpu_sc as plsc

B = 4096
DENSE_IN = 13
F = 26
VOCAB = 100000
D = 32
N1 = F + 1
NPAD = 32
BF = B * F

NW = 32
ROWS_PER_W = BF // NW
CHUNK = 128
NCH = ROWS_PER_W // CHUNK

BT = 512
NBLK = B // BT
DPAD = 16

_PI, _PJ = np.triu_indices(N1, k=1)


def _sc_gather(table_flat, idx2d):
    mesh = plsc.VectorSubcoreMesh(core_axis_name="c", subcore_axis_name="s")

    @functools.partial(
        pl.kernel,
        mesh=mesh,
        out_type=jax.ShapeDtypeStruct((BF, D), jnp.float32),
        scratch_types=[
            pltpu.VMEM((NCH, CHUNK), jnp.int32),
            pltpu.VMEM((ROWS_PER_W, D), jnp.float32),
            pltpu.SemaphoreType.DMA,
        ],
        compiler_params=pltpu.CompilerParams(use_tc_tiling_on_sc=False),
    )
    def gather_kernel(table_hbm, idx_hbm, out_hbm, idx_v, rows_v, sem):
        wid = lax.axis_index("s") * 2 + lax.axis_index("c")
        pltpu.sync_copy(idx_hbm.at[wid], idx_v)
        copies = []
        for j in range(NCH):
            copies.append(
                pltpu.async_copy(
                    table_hbm.at[idx_v.at[j]],
                    rows_v.at[pl.ds(j * CHUNK, CHUNK)],
                    sem,
                )
            )
        for c in copies:
            c.wait()
        pltpu.sync_copy(rows_v, out_hbm.at[pl.ds(wid * ROWS_PER_W, ROWS_PER_W)])

    return gather_kernel(table_flat, idx2d)


def _tc_body(dense_ref, pooled_ref, lab_ref,
             dW0_ref, dB0_ref, dW1_ref, dB1_ref, dW2_ref, dB2_ref,
             oW0a_ref, Wp_ref, oB0_ref, oW1_ref, oB1_ref, oW2_ref, oB2_ref,
             probs_ref, loss_ref):
    x = dense_ref[...]
    x = jnp.maximum(dW0_ref[...] @ x + dB0_ref[...], 0.0)
    x = jnp.maximum(dW1_ref[...] @ x + dB1_ref[...], 0.0)
    e = jnp.maximum(dW2_ref[...] @ x + dB2_ref[...], 0.0)

    P = jnp.concatenate(
        [e, pooled_ref[...], jnp.zeros(((NPAD - N1) * D, BT), jnp.float32)],
        axis=0).reshape(NPAD, D, BT)

    rows = []
    for n in range(N1):
        an = P[n]
        rows.append(jnp.sum(P * an[None], axis=1))
    R = jnp.concatenate(rows, axis=0)

    t = oW0a_ref[...] @ e + Wp_ref[...] @ R + oB0_ref[...]
    h = jnp.maximum(t, 0.0)
    h = jnp.maximum(oW1_ref[...] @ h + oB1_ref[...], 0.0)
    z = oW2_ref[...] @ h + oB2_ref[...]

    pr = 1.0 / (1.0 + jnp.exp(-z))
    probs_ref[...] = pr

    p = jnp.clip(pr, 1e-7, 1.0 - 1e-7)
    lab = lab_ref[...]
    s = jnp.sum(lab * jnp.log(p) + (1.0 - lab) * jnp.log1p(-p),
                keepdims=True)
    contrib = -s * (1.0 / B)

    i = pl.program_id(0)

    @pl.when(i == 0)
    def _():
        loss_ref[...] = contrib

    @pl.when(i > 0)
    def _():
        loss_ref[...] += contrib


def _tc_forward(dense_t, pooled_t, labf, dW0t, dB0, dW1t, dB1, dW2t, dB2,
                oW0at, Wpt, oB0, oW1t, oB1, oW2t, oB2, interpret=False):
    full = lambda a: pl.BlockSpec(a.shape, lambda i: (0, 0))
    grid = (NBLK,)
    probs, loss = pl.pallas_call(
        _tc_body,
        grid=grid,
        in_specs=[
            pl.BlockSpec((DPAD, BT), lambda i: (0, i)),
            pl.BlockSpec((F * D, BT), lambda i: (0, i)),
            pl.BlockSpec((1, BT), lambda i: (0, i)),
            full(dW0t), full(dB0), full(dW1t), full(dB1), full(dW2t),
            full(dB2), full(oW0at), full(Wpt), full(oB0), full(oW1t),
            full(oB1), full(oW2t), full(oB2),
        ],
        out_specs=[
            pl.BlockSpec((1, BT), lambda i: (0, i)),
            pl.BlockSpec((1, 1), lambda i: (0, 0)),
        ],
        out_shape=[
            jax.ShapeDtypeStruct((1, B), jnp.float32),
            jax.ShapeDtypeStruct((1, 1), jnp.float32),
        ],
        interpret=interpret,
    )(dense_t, pooled_t, labf, dW0t, dB0, dW1t, dB1, dW2t, dB2,
      oW0at, Wpt, oB0, oW1t, oB1, oW2t, oB2)
    return probs, loss


def _fold_pair_weights(oW0):
    H = oW0.shape[1]
    oW0p = oW0[D:]
    W2 = jnp.zeros((N1, N1, H), jnp.float32)
    W2 = W2.at[_PI, _PJ].set(0.5 * oW0p).at[_PJ, _PI].set(0.5 * oW0p)
    W2 = jnp.pad(W2, ((0, 0), (0, NPAD - N1), (0, 0)))
    return W2.reshape(N1 * NPAD, H)


def kernel(dense_features, sparse_indices, labels, tables,
           dW0, dB0, dW1, dB1, dW2, dB2,
           oW0, oB0, oW1, oB1, oW2, oB2):
    table_flat = tables.reshape(F * VOCAB, D)
    flat_idx = (sparse_indices
                + (jnp.arange(F, dtype=jnp.int32) * VOCAB)[None, :])
    idx2d = flat_idx.reshape(NW, NCH, CHUNK)

    pooled = _sc_gather(table_flat, idx2d)
    pooled_t = pooled.reshape(B, F * D).T

    dense_t = jnp.pad(dense_features, ((0, 0), (0, DPAD - DENSE_IN))).T
    dW0tp = jnp.pad(dW0.T, ((0, 0), (0, DPAD - DENSE_IN)))

    Wpt = _fold_pair_weights(oW0).T
    probs, loss = _tc_forward(
        dense_t, pooled_t,
        labels.astype(jnp.float32).reshape(1, B),
        dW0tp, dB0.reshape(-1, 1), dW1.T, dB1.reshape(-1, 1),
        dW2.T, dB2.reshape(-1, 1),
        oW0[:D].T, Wpt, oB0.reshape(-1, 1), oW1.T, oB1.reshape(-1, 1),
        oW2.T, oB2.reshape(1, 1))
    return (loss[0, 0], probs.reshape(B), labels)


# speedup vs baseline: 2.2714x; 1.0393x over previous
import functools
import numpy as np
import jax
import jax.numpy as jnp
from jax import lax
from jax.experimental import pallas as pl
from jax.experimental.pallas import tpu as pltpu
from jax.experimental.pallas import tpu_sc as plsc

B = 4096
DENSE_IN = 13
F = 26
VOCAB = 100000
D = 32
N1 = F + 1
NPAD = 32
BF = B * F

NW = 32
ROWS_PER_W = BF // NW
CHUNK = 128
NCH = ROWS_PER_W // CHUNK

BT = 512
NBLK = B // BT
DPAD = 16

_PI, _PJ = np.triu_indices(N1, k=1)


def _sc_gather(table_flat, idx2d):
    mesh = plsc.VectorSubcoreMesh(core_axis_name="c", subcore_axis_name="s")

    @functools.partial(
        pl.kernel,
        mesh=mesh,
        out_type=jax.ShapeDtypeStruct((BF, D), jnp.float32),
        scratch_types=[
            pltpu.VMEM((NCH, CHUNK), jnp.int32),
            pltpu.VMEM((ROWS_PER_W, D), jnp.float32),
            pltpu.SemaphoreType.DMA,
        ],
        compiler_params=pltpu.CompilerParams(use_tc_tiling_on_sc=False),
    )
    def gather_kernel(table_hbm, idx_hbm, out_hbm, idx_v, rows_v, sem):
        wid = lax.axis_index("s") * 2 + lax.axis_index("c")
        pltpu.sync_copy(idx_hbm.at[wid], idx_v)
        copies = []
        for j in range(NCH):
            copies.append(
                pltpu.async_copy(
                    table_hbm.at[idx_v.at[j]],
                    rows_v.at[pl.ds(j * CHUNK, CHUNK)],
                    sem,
                )
            )
        for c in copies:
            c.wait()
        pltpu.sync_copy(rows_v, out_hbm.at[pl.ds(wid * ROWS_PER_W, ROWS_PER_W)])

    return gather_kernel(table_flat, idx2d)


def _tc_body(dense_ref, pooled_ref, lab_ref,
             dW0_ref, dB0_ref, dW1_ref, dB1_ref, dW2_ref, dB2_ref,
             oW0a_ref, Wp_ref, oB0_ref, oW1_ref, oB1_ref, oW2_ref, oB2_ref,
             probs_ref, loss_ref):
    x = dense_ref[...]
    x = jnp.maximum(dW0_ref[...] @ x + dB0_ref[...], 0.0)
    x = jnp.maximum(dW1_ref[...] @ x + dB1_ref[...], 0.0)
    e = jnp.maximum(dW2_ref[...] @ x + dB2_ref[...], 0.0)

    pooled_t = pooled_ref[...].T
    P = jnp.concatenate(
        [e, pooled_t, jnp.zeros(((NPAD - N1) * D, BT), jnp.float32)],
        axis=0).reshape(NPAD, D, BT)

    rows = []
    for n in range(N1):
        an = P[n]
        rows.append(jnp.sum(P * an[None], axis=1))
    R = jnp.concatenate(rows, axis=0)

    t = oW0a_ref[...] @ e + Wp_ref[...] @ R + oB0_ref[...]
    h = jnp.maximum(t, 0.0)
    h = jnp.maximum(oW1_ref[...] @ h + oB1_ref[...], 0.0)
    z = oW2_ref[...] @ h + oB2_ref[...]

    pr = 1.0 / (1.0 + jnp.exp(-z))
    probs_ref[...] = pr

    p = jnp.clip(pr, 1e-7, 1.0 - 1e-7)
    lab = lab_ref[...]
    s = jnp.sum(lab * jnp.log(p) + (1.0 - lab) * jnp.log1p(-p),
                keepdims=True)
    contrib = -s * (1.0 / B)

    i = pl.program_id(0)

    @pl.when(i == 0)
    def _():
        loss_ref[...] = contrib

    @pl.when(i > 0)
    def _():
        loss_ref[...] += contrib


def _tc_forward(dense_t, pooled_t, labf, dW0t, dB0, dW1t, dB1, dW2t, dB2,
                oW0at, Wpt, oB0, oW1t, oB1, oW2t, oB2, interpret=False):
    full = lambda a: pl.BlockSpec(a.shape, lambda i: (0, 0))
    grid = (NBLK,)
    probs, loss = pl.pallas_call(
        _tc_body,
        grid=grid,
        in_specs=[
            pl.BlockSpec((DPAD, BT), lambda i: (0, i)),
            pl.BlockSpec((BT, F * D), lambda i: (i, 0)),
            pl.BlockSpec((1, BT), lambda i: (0, i)),
            full(dW0t), full(dB0), full(dW1t), full(dB1), full(dW2t),
            full(dB2), full(oW0at), full(Wpt), full(oB0), full(oW1t),
            full(oB1), full(oW2t), full(oB2),
        ],
        out_specs=[
            pl.BlockSpec((1, BT), lambda i: (0, i)),
            pl.BlockSpec((1, 1), lambda i: (0, 0)),
        ],
        out_shape=[
            jax.ShapeDtypeStruct((1, B), jnp.float32),
            jax.ShapeDtypeStruct((1, 1), jnp.float32),
        ],
        interpret=interpret,
    )(dense_t, pooled_t, labf, dW0t, dB0, dW1t, dB1, dW2t, dB2,
      oW0at, Wpt, oB0, oW1t, oB1, oW2t, oB2)
    return probs, loss


_NPAIR = N1 * F // 2
_PAIR_OF = np.full((N1, NPAD), _NPAIR, np.int32)
_PAIR_OF[_PI, _PJ] = np.arange(_NPAIR)
_PAIR_OF[_PJ, _PI] = np.arange(_NPAIR)
_PAIR_GIDX = _PAIR_OF.reshape(-1)


def _fold_pair_weights_t(oW0):
    H = oW0.shape[1]
    ext = jnp.concatenate([0.5 * oW0[D:].T, jnp.zeros((H, 1), jnp.float32)],
                          axis=1)
    return ext[:, _PAIR_GIDX]


def kernel(dense_features, sparse_indices, labels, tables,
           dW0, dB0, dW1, dB1, dW2, dB2,
           oW0, oB0, oW1, oB1, oW2, oB2):
    table_flat = tables.reshape(F * VOCAB, D)
    flat_idx = (sparse_indices
                + (jnp.arange(F, dtype=jnp.int32) * VOCAB)[None, :])
    idx2d = flat_idx.reshape(NW, NCH, CHUNK)

    pooled = _sc_gather(table_flat, idx2d)
    pooled2d = pooled.reshape(B, F * D)

    dense_t = jnp.pad(dense_features, ((0, 0), (0, DPAD - DENSE_IN))).T
    dW0tp = jnp.pad(dW0.T, ((0, 0), (0, DPAD - DENSE_IN)))

    Wpt = _fold_pair_weights_t(oW0)
    probs, loss = _tc_forward(
        dense_t, pooled2d,
        labels.astype(jnp.float32).reshape(1, B),
        dW0tp, dB0.reshape(-1, 1), dW1.T, dB1.reshape(-1, 1),
        dW2.T, dB2.reshape(-1, 1),
        oW0[:D].T, Wpt, oB0.reshape(-1, 1), oW1.T, oB1.reshape(-1, 1),
        oW2.T, oB2.reshape(1, 1))
    return (loss[0, 0], probs.reshape(B), labels)
